# baseline (device time: 18891 ns/iter reference)
import jax
import jax.numpy as jnp
from jax import lax
from jax.experimental import pallas as pl
from jax.experimental.pallas import tpu as pltpu

N_DEV = 4


def kernel(x, W1, W2):
    m, k = x.shape
    n = W2.shape[1]
    mq = m // N_DEV

    def body(x_ref, w1_ref, w2_ref, out_ref,
             part_buf, rs_buf, ag_src,
             rs_send, rs_recv, ag_send, ag_recv):
        my = lax.axis_index("i")

        barrier_sem = pltpu.get_barrier_semaphore()
        for t in range(N_DEV - 1):
            pl.semaphore_signal(
                barrier_sem, inc=1,
                device_id=(lax.rem(my + 1 + t, N_DEV),),
                device_id_type=pl.DeviceIdType.MESH,
            )

        w1b = w1_ref[...].astype(jnp.bfloat16)
        w2b = w2_ref[...].astype(jnp.bfloat16)

        def quarter(dst):
            xc = x_ref[pl.ds(dst * mq, mq), :].astype(jnp.bfloat16)
            hc = jnp.dot(xc, w1b, preferred_element_type=jnp.float32)
            hc = jnp.maximum(hc, 0.0).astype(jnp.bfloat16)
            return jnp.dot(hc, w2b, preferred_element_type=jnp.float32)

        pl.semaphore_wait(barrier_sem, N_DEV - 1)

        rs = []
        for t in range(N_DEV - 1):
            dst = lax.rem(my + 1 + t, N_DEV)
            part_buf[t] = quarter(dst).astype(jnp.bfloat16)
            rdma = pltpu.make_async_remote_copy(
                src_ref=part_buf.at[t],
                dst_ref=rs_buf.at[2 - t],
                send_sem=rs_send.at[t],
                recv_sem=rs_recv.at[2 - t],
                device_id=(dst,),
                device_id_type=pl.DeviceIdType.MESH,
            )
            rdma.start()
            rs.append(rdma)

        red = quarter(my)
        for rdma in rs:
            rdma.wait()
        for s in range(N_DEV - 1):
            red = red + rs_buf[s].astype(jnp.float32)
        redb = red.astype(jnp.bfloat16)
        out_ref[pl.ds(my * mq, mq), :] = redb
        ag_src[...] = redb

        ag = []
        for t in range(N_DEV - 1):
            dst = lax.rem(my + 1 + t, N_DEV)
            rdma = pltpu.make_async_remote_copy(
                src_ref=ag_src,
                dst_ref=out_ref.at[pl.ds(my * mq, mq), :],
                send_sem=ag_send.at[t],
                recv_sem=ag_recv.at[2 - t],
                device_id=(dst,),
                device_id_type=pl.DeviceIdType.MESH,
            )
            rdma.start()
            ag.append(rdma)
        for rdma in ag:
            rdma.wait()

    return pl.pallas_call(
        body,
        out_shape=jax.ShapeDtypeStruct((m, n), jnp.bfloat16),
        in_specs=[
            pl.BlockSpec(memory_space=pltpu.VMEM),
            pl.BlockSpec(memory_space=pltpu.VMEM),
            pl.BlockSpec(memory_space=pltpu.VMEM),
        ],
        out_specs=pl.BlockSpec(memory_space=pltpu.VMEM),
        scratch_shapes=[
            pltpu.VMEM((N_DEV - 1, mq, n), jnp.bfloat16),
            pltpu.VMEM((N_DEV - 1, mq, n), jnp.bfloat16),
            pltpu.VMEM((mq, n), jnp.bfloat16),
            pltpu.SemaphoreType.DMA((N_DEV - 1,)),
            pltpu.SemaphoreType.DMA((N_DEV - 1,)),
            pltpu.SemaphoreType.DMA((N_DEV - 1,)),
            pltpu.SemaphoreType.DMA((N_DEV - 1,)),
        ],
        compiler_params=pltpu.CompilerParams(collective_id=0),
    )(x, W1, W2)
